# Initial kernel scaffold; baseline (speedup 1.0000x reference)
import jax
import jax.numpy as jnp
from jax.experimental import pallas as pl


def kernel(x, edge_feats, node_logprobs, edge_logprobs, max_bonds_table, edge_index, isgen, graph_ids):
    raise NotImplementedError("write your pallas kernel here")


# speedup vs baseline: 9.7762x; 9.7762x over previous
import functools

import jax
import jax.numpy as jnp
from jax import lax
from jax.experimental import pallas as pl
from jax.experimental.pallas import tpu as pltpu
from jax.experimental.pallas import tpu_sc as plsc

N = 10000
E = 320000
D = 128
G = 256

NP = 10240
NW = 32
EPW = E // NW
WIN = 2000
NVREG = WIN // 16
NWIN = EPW // WIN
ACCW = 4 * NP
RED = ACCW // 16

NB = 2048
NGRID = (N + NB - 1) // NB
EB = 4096
EGRID = (E + EB - 1) // EB


def _node_feats_body(x_ref, gid_ref, tab_ref, is42_ref, chg_ref, arm_ref,
                     mxb_ref, cnt_ref):
    i = pl.program_id(0)
    xb = x_ref[...]
    lane = lax.broadcasted_iota(jnp.int32, (NB, D), 1)

    neg = jnp.float32(-3.4e38)
    m_at = lane < 43
    mx = jnp.max(jnp.where(m_at, xb, neg), axis=1, keepdims=True)
    cand = jnp.where((xb == mx) & m_at, lane, D)
    atoms = jnp.min(cand, axis=1)
    is42 = (atoms == 42).astype(jnp.float32)
    is42_ref[...] = is42

    m_ch = (lane >= 43) & (lane < 50)
    mx2 = jnp.max(jnp.where(m_ch, xb, neg), axis=1, keepdims=True)
    cand2 = jnp.where((xb == mx2) & m_ch, lane, D)
    chg_ref[...] = (jnp.min(cand2, axis=1) - 46).astype(jnp.float32)

    arm_ref[...] = (xb[:, 127] > xb[:, 126]).astype(jnp.float32)

    tab = tab_ref[...].reshape(1, D)
    onehot = lane == atoms[:, None]
    mxb_ref[...] = jnp.sum(jnp.where(onehot, tab, 0.0), axis=1)

    gid = gid_ref[...]
    gi = lax.broadcasted_iota(jnp.int32, (NB, G), 1)
    rowpos = lax.broadcasted_iota(jnp.int32, (NB, G), 0)
    valid = rowpos + i * NB < N
    goh = (gid[:, None] == gi) & valid
    part = jnp.sum(goh.astype(jnp.float32), axis=0)

    @pl.when(i == 0)
    def _():
        cnt_ref[...] = jnp.zeros_like(cnt_ref)

    cnt_ref[...] += part


def _node_feats(x, graph_ids, tab_pad):
    out = [jax.ShapeDtypeStruct((N,), jnp.float32) for _ in range(4)]
    out.append(jax.ShapeDtypeStruct((G,), jnp.float32))
    return pl.pallas_call(
        _node_feats_body,
        grid=(NGRID,),
        in_specs=[
            pl.BlockSpec((NB, D), lambda i: (i, 0)),
            pl.BlockSpec((NB,), lambda i: (i,)),
            pl.BlockSpec((D,), lambda i: (0,)),
        ],
        out_specs=[
            pl.BlockSpec((NB,), lambda i: (i,)),
            pl.BlockSpec((NB,), lambda i: (i,)),
            pl.BlockSpec((NB,), lambda i: (i,)),
            pl.BlockSpec((NB,), lambda i: (i,)),
            pl.BlockSpec((G,), lambda i: (0,)),
        ],
        out_shape=out,
    )(x, graph_ids, tab_pad)


def _edge_feats_body(ef_ref, ig_ref, b2_ref, b4_ref):
    ef = ef_ref[...]
    lane = lax.broadcasted_iota(jnp.int32, (EB, 16), 1)
    neg = jnp.float32(-3.4e38)
    m5 = lane < 5
    mx = jnp.max(jnp.where(m5, ef, neg), axis=1, keepdims=True)
    cand = jnp.where((ef == mx) & m5, lane, 16)
    bt = jnp.min(cand, axis=1)
    b4 = (bt == 4).astype(jnp.float32)
    b2 = jnp.where(bt == 4, 1, bt).astype(jnp.float32)
    b2 = jnp.where(ig_ref[...] == -1, 0.0, b2)
    b2_ref[...] = b2
    b4_ref[...] = b4


def _edge_feats(edge_feats, isgen):
    return pl.pallas_call(
        _edge_feats_body,
        grid=(EGRID,),
        in_specs=[
            pl.BlockSpec((EB, 16), lambda i: (i, 0)),
            pl.BlockSpec((EB,), lambda i: (i,)),
        ],
        out_specs=[
            pl.BlockSpec((EB,), lambda i: (i,)),
            pl.BlockSpec((EB,), lambda i: (i,)),
        ],
        out_shape=[jax.ShapeDtypeStruct((E,), jnp.float32) for _ in range(2)],
    )(edge_feats, isgen)


def _sc_body(src_hbm, dst_hbm, lp_hbm, b2_hbm, b4_hbm, is42_hbm, out_hbm,
             is42_v, srcb, dstb, lpb, b2b, b4b, acc, res, tmp, spmem):
    core = lax.axis_index("c")
    sub = lax.axis_index("s")
    wid = sub * 2 + core
    ebase = wid * EPW

    def zero_body(j, _):
        acc[pl.ds(j * 16, 16)] = jnp.zeros((16,), jnp.float32)
        return 0

    lax.fori_loop(0, ACCW // 16, zero_body, 0)

    pltpu.sync_copy(is42_hbm, is42_v.at[pl.ds(0, N)])

    ones16 = jnp.ones((16,), jnp.float32)

    for w in range(NWIN):
        off = ebase + w * WIN
        pltpu.sync_copy(src_hbm.at[pl.ds(off, WIN)], srcb)
        pltpu.sync_copy(dst_hbm.at[pl.ds(off, WIN)], dstb)
        pltpu.sync_copy(lp_hbm.at[pl.ds(off, WIN)], lpb)
        pltpu.sync_copy(b2_hbm.at[pl.ds(off, WIN)], b2b)
        pltpu.sync_copy(b4_hbm.at[pl.ds(off, WIN)], b4b)

        def vreg_body(v, _):
            o = v * 16
            s = srcb[pl.ds(o, 16)]
            d = dstb[pl.ds(o, 16)]
            s42 = plsc.load_gather(is42_v, [s])
            d42 = plsc.load_gather(is42_v, [d])
            b4 = b4b[pl.ds(o, 16)]
            b2 = b2b[pl.ds(o, 16)]
            lp = lpb[pl.ds(o, 16)]
            arom = b4 * (1.0 - s42) * (1.0 - d42)
            bond = b2 * (1.0 - s42)
            cnt, _ = plsc.scan_count(d)
            maxc = jnp.max(cnt)

            def round_body(r, _):
                mk = cnt == r
                plsc.addupdate_scatter(acc, [d], arom, mask=mk)
                plsc.addupdate_scatter(acc, [d + NP], bond, mask=mk)
                plsc.addupdate_scatter(acc, [d + 2 * NP], ones16, mask=mk)
                plsc.addupdate_scatter(acc, [d + 3 * NP], lp, mask=mk)
                return 0

            lax.fori_loop(0, maxc + 1, round_body, 0)
            return 0

        lax.fori_loop(0, NVREG, vreg_body, 0)

    pltpu.sync_copy(acc, spmem.at[sub])
    plsc.subcore_barrier()

    rbase = sub * RED
    pltpu.sync_copy(spmem.at[0, pl.ds(rbase, RED)], res)

    def red_body(i, _):
        pltpu.sync_copy(spmem.at[i, pl.ds(rbase, RED)], tmp)

        def add_body(j, _):
            sl = pl.ds(j * 16, 16)
            res[sl] += tmp[sl]
            return 0

        lax.fori_loop(0, RED // 16, add_body, 0)
        return 0

    lax.fori_loop(1, 16, red_body, 0)

    pltpu.sync_copy(res, out_hbm.at[core, pl.ds(rbase, RED)])


def _sc_edge_aggregate(src, dst, elogp, b2, b4, is42):
    mesh = plsc.VectorSubcoreMesh(core_axis_name="c", subcore_axis_name="s")
    f = pl.kernel(
        _sc_body,
        mesh=mesh,
        compiler_params=pltpu.CompilerParams(needs_layout_passes=False),
        out_type=jax.ShapeDtypeStruct((2, ACCW), jnp.float32),
        scratch_types=[
            pltpu.VMEM((NP,), jnp.float32),
            pltpu.VMEM((WIN,), jnp.int32),
            pltpu.VMEM((WIN,), jnp.int32),
            pltpu.VMEM((WIN,), jnp.float32),
            pltpu.VMEM((WIN,), jnp.float32),
            pltpu.VMEM((WIN,), jnp.float32),
            pltpu.VMEM((ACCW,), jnp.float32),
            pltpu.VMEM((RED,), jnp.float32),
            pltpu.VMEM((RED,), jnp.float32),
            pltpu.VMEM_SHARED((16, ACCW), jnp.float32),
        ],
    )
    return f(src, dst, elogp, b2, b4, is42)


def _finalize_body(sums_ref, is42_ref, chg_ref, arm_ref, mxb_ref, nlp_ref,
                   gid_ref, cnt_ref, tl_ref, tr_ref, sl_acc, sr_acc):
    i = pl.program_id(0)

    def row(k):
        return jnp.sum(sums_ref[k:k + 1, :], axis=0)

    arom_sum = row(0) + row(4)
    bonds = row(1) + row(5)
    deg = row(2) + row(6)
    selp = row(3) + row(7)

    is42 = is42_ref[...] > 0.5
    charges = chg_ref[...]
    aromflag = arm_ref[...] > 0.5
    maxbonds = mxb_ref[...]
    nlp = nlp_ref[...]
    gid = gid_ref[...]
    counts = cnt_ref[...]

    has_arom = arom_sum > 0.0
    af = (has_arom != aromflag).astype(jnp.float32)
    valency = bonds - charges
    vf = (valency > maxbonds).astype(jnp.float32)
    bf = (is42 != (bonds == 0.0)).astype(jnp.float32)
    reward = -1.0 * af - 2.0 * vf - 3.0 * bf

    gi = lax.broadcasted_iota(jnp.int32, (NB, G), 1)
    rowpos = lax.broadcasted_iota(jnp.int32, (NB, G), 0)
    valid = rowpos + i * NB < N
    goh = gid[:, None] == gi
    cb = counts.reshape(1, G)
    nn = jnp.sum(jnp.where(goh, cb, 0.0), axis=1)
    reward = jnp.where((nn == 1.0) & is42, -4.0, reward)

    selp_mean = selp / jnp.maximum(deg, 1.0)
    loss = -(nlp + selp_mean) * reward

    gohv = goh & valid
    ploss = jnp.sum(jnp.where(gohv, loss[:, None], 0.0), axis=0)
    prew = jnp.sum(jnp.where(gohv, reward[:, None], 0.0), axis=0)

    @pl.when(i == 0)
    def _():
        sl_acc[...] = jnp.zeros_like(sl_acc)
        sr_acc[...] = jnp.zeros_like(sr_acc)

    sl_acc[...] += ploss
    sr_acc[...] += prew

    @pl.when(i == NGRID - 1)
    def _():
        c = jnp.maximum(counts, 1.0)
        tl_ref[...] = (jnp.sum(sl_acc[...] / c) / G).reshape(1, 1)
        tr_ref[...] = (jnp.sum(sr_acc[...] / c) / G).reshape(1, 1)


def _finalize(sums8, is42, chg, arm, mxb, nlp, graph_ids, counts):
    return pl.pallas_call(
        _finalize_body,
        grid=(NGRID,),
        in_specs=[
            pl.BlockSpec((8, NB), lambda i: (0, i)),
            pl.BlockSpec((NB,), lambda i: (i,)),
            pl.BlockSpec((NB,), lambda i: (i,)),
            pl.BlockSpec((NB,), lambda i: (i,)),
            pl.BlockSpec((NB,), lambda i: (i,)),
            pl.BlockSpec((NB,), lambda i: (i,)),
            pl.BlockSpec((NB,), lambda i: (i,)),
            pl.BlockSpec((G,), lambda i: (0,)),
        ],
        out_specs=[
            pl.BlockSpec((1, 1), lambda i: (0, 0)),
            pl.BlockSpec((1, 1), lambda i: (0, 0)),
        ],
        out_shape=[jax.ShapeDtypeStruct((1, 1), jnp.float32) for _ in range(2)],
        scratch_shapes=[
            pltpu.VMEM((G,), jnp.float32),
            pltpu.VMEM((G,), jnp.float32),
        ],
    )(sums8, is42, chg, arm, mxb, nlp, graph_ids, counts)


@jax.jit
def kernel(x, edge_feats, node_logprobs, edge_logprobs, max_bonds_table,
           edge_index, isgen, graph_ids):
    tab_pad = jnp.zeros((D,), jnp.float32).at[:43].set(max_bonds_table)
    is42, chg, arm, mxb, counts = _node_feats(x, graph_ids, tab_pad)
    b2, b4 = _edge_feats(edge_feats, isgen)
    src = edge_index[0]
    dst = edge_index[1]
    parts = _sc_edge_aggregate(src, dst, edge_logprobs, b2, b4, is42)
    sums = parts.reshape(2, 4, NP)[:, :, :N].reshape(8, N)
    tl, tr = _finalize(sums, is42, chg, arm, mxb, node_logprobs, graph_ids,
                       counts)
    return (tl[0, 0], tr[0, 0])


# speedup vs baseline: 23.4821x; 2.4020x over previous
import functools

import jax
import jax.numpy as jnp
from jax import lax
from jax.experimental import pallas as pl
from jax.experimental.pallas import tpu as pltpu
from jax.experimental.pallas import tpu_sc as plsc

N = 10000
E = 320000
D = 128
G = 256

NP = 10240
NW = 32
EPW = E // NW
WIN = 400
NVREG = WIN // 16
NWIN = EPW // WIN
ACCW = 4 * NP
RED = ACCW // 16

NB = 2048
NGRID = (N + NB - 1) // NB
EB = 4096
EGRID = (E + EB - 1) // EB


def _node_feats_body(x_ref, gid_ref, tab_ref, is42_ref, chg_ref, arm_ref,
                     mxb_ref, cnt_ref):
    i = pl.program_id(0)
    xb = x_ref[...]
    lane = lax.broadcasted_iota(jnp.int32, (NB, D), 1)

    neg = jnp.float32(-3.4e38)
    m_at = lane < 43
    mx = jnp.max(jnp.where(m_at, xb, neg), axis=1, keepdims=True)
    cand = jnp.where((xb == mx) & m_at, lane, D)
    atoms = jnp.min(cand, axis=1)
    is42 = (atoms == 42).astype(jnp.float32)
    is42_ref[...] = is42

    m_ch = (lane >= 43) & (lane < 50)
    mx2 = jnp.max(jnp.where(m_ch, xb, neg), axis=1, keepdims=True)
    cand2 = jnp.where((xb == mx2) & m_ch, lane, D)
    chg_ref[...] = (jnp.min(cand2, axis=1) - 46).astype(jnp.float32)

    arm_ref[...] = (xb[:, 127] > xb[:, 126]).astype(jnp.float32)

    tab = tab_ref[...].reshape(1, D)
    onehot = lane == atoms[:, None]
    mxb_ref[...] = jnp.sum(jnp.where(onehot, tab, 0.0), axis=1)

    gid = gid_ref[...]
    gi = lax.broadcasted_iota(jnp.int32, (NB, G), 1)
    rowpos = lax.broadcasted_iota(jnp.int32, (NB, G), 0)
    valid = rowpos + i * NB < N
    goh = (gid[:, None] == gi) & valid
    part = jnp.sum(goh.astype(jnp.float32), axis=0)

    @pl.when(i == 0)
    def _():
        cnt_ref[...] = jnp.zeros_like(cnt_ref)

    cnt_ref[...] += part


def _node_feats(x, graph_ids, tab_pad):
    out = [jax.ShapeDtypeStruct((N,), jnp.float32) for _ in range(4)]
    out.append(jax.ShapeDtypeStruct((G,), jnp.float32))
    return pl.pallas_call(
        _node_feats_body,
        grid=(NGRID,),
        in_specs=[
            pl.BlockSpec((NB, D), lambda i: (i, 0)),
            pl.BlockSpec((NB,), lambda i: (i,)),
            pl.BlockSpec((D,), lambda i: (0,)),
        ],
        out_specs=[
            pl.BlockSpec((NB,), lambda i: (i,)),
            pl.BlockSpec((NB,), lambda i: (i,)),
            pl.BlockSpec((NB,), lambda i: (i,)),
            pl.BlockSpec((NB,), lambda i: (i,)),
            pl.BlockSpec((G,), lambda i: (0,)),
        ],
        out_shape=out,
    )(x, graph_ids, tab_pad)


def _sc_body(src_hbm, dst_hbm, lp_hbm, ef_hbm, ig_hbm, is42_hbm, out_hbm,
             is42_v, srcb, dstb, lpb, igb, efb, acc, res, tmp, spmem, sems):
    core = lax.axis_index("c")
    sub = lax.axis_index("s")
    wid = sub * 2 + core
    ebase = wid * EPW

    def zero_body(j, _):
        acc[pl.ds(j * 16, 16)] = jnp.zeros((16,), jnp.float32)
        return 0

    lax.fori_loop(0, ACCW // 16, zero_body, 0)

    pltpu.sync_copy(is42_hbm, is42_v.at[pl.ds(0, N)])

    ones16 = jnp.ones((16,), jnp.float32)

    def issue(w, bi):
        off = ebase + w * WIN
        sl = pl.ds(bi * WIN, WIN)
        sem = sems.at[bi]
        return [
            pltpu.async_copy(src_hbm.at[pl.ds(off, WIN)], srcb.at[sl], sem),
            pltpu.async_copy(dst_hbm.at[pl.ds(off, WIN)], dstb.at[sl], sem),
            pltpu.async_copy(lp_hbm.at[pl.ds(off, WIN)], lpb.at[sl], sem),
            pltpu.async_copy(ig_hbm.at[pl.ds(off, WIN)], igb.at[sl], sem),
            pltpu.async_copy(ef_hbm.at[pl.ds(off * 16, WIN * 16)],
                             efb.at[pl.ds(bi * WIN * 16, WIN * 16)], sem),
        ]

    pending = {0: issue(0, 0)}

    for w in range(NWIN):
        bi = w % 2
        if w + 1 < NWIN:
            pending[(w + 1) % 2] = issue(w + 1, (w + 1) % 2)
        for h in pending[bi]:
            h.wait()
        ob = bi * WIN

        def vreg_body(v, _):
            o = ob + v * 16
            s = srcb[pl.ds(o, 16)]
            d = dstb[pl.ds(o, 16)]
            s42 = plsc.load_gather(is42_v, [s])
            d42 = plsc.load_gather(is42_v, [d])
            base16 = (lax.iota(jnp.int32, 16) + o) * 16
            m = plsc.load_gather(efb, [base16])
            b = jnp.zeros((16,), jnp.float32)
            for j in range(1, 5):
                cj = plsc.load_gather(efb, [base16 + j])
                gt = cj > m
                b = jnp.where(gt, jnp.float32(j), b)
                m = jnp.where(gt, cj, m)
            b4 = (b == 4.0).astype(jnp.float32)
            b2 = jnp.where(b == 4.0, 1.0, b)
            b2 = jnp.where(igb[pl.ds(o, 16)] == -1, 0.0, b2)
            lp = lpb[pl.ds(o, 16)]
            arom = b4 * (1.0 - s42) * (1.0 - d42)
            bond = b2 * (1.0 - s42)
            cnt, _ = plsc.scan_count(d)
            maxc = jnp.max(cnt)

            def round_body(r, _):
                mk = cnt == r
                plsc.addupdate_scatter(acc, [d], arom, mask=mk)
                plsc.addupdate_scatter(acc, [d + NP], bond, mask=mk)
                plsc.addupdate_scatter(acc, [d + 2 * NP], ones16, mask=mk)
                plsc.addupdate_scatter(acc, [d + 3 * NP], lp, mask=mk)
                return 0

            lax.fori_loop(0, maxc + 1, round_body, 0)
            return 0

        lax.fori_loop(0, NVREG, vreg_body, 0)

    pltpu.sync_copy(acc, spmem.at[sub])
    plsc.subcore_barrier()

    rbase = sub * RED
    pltpu.sync_copy(spmem.at[0, pl.ds(rbase, RED)], res)

    def red_body(i, _):
        pltpu.sync_copy(spmem.at[i, pl.ds(rbase, RED)], tmp)

        def add_body(j, _):
            sl = pl.ds(j * 16, 16)
            res[sl] += tmp[sl]
            return 0

        lax.fori_loop(0, RED // 16, add_body, 0)
        return 0

    lax.fori_loop(1, 16, red_body, 0)

    pltpu.sync_copy(res, out_hbm.at[core, pl.ds(rbase, RED)])


def _sc_edge_aggregate(src, dst, elogp, edge_feats, isgen, is42):
    mesh = plsc.VectorSubcoreMesh(core_axis_name="c", subcore_axis_name="s")
    f = pl.kernel(
        _sc_body,
        mesh=mesh,
        compiler_params=pltpu.CompilerParams(needs_layout_passes=False),
        out_type=jax.ShapeDtypeStruct((2, ACCW), jnp.float32),
        scratch_types=[
            pltpu.VMEM((NP,), jnp.float32),
            pltpu.VMEM((2 * WIN,), jnp.int32),
            pltpu.VMEM((2 * WIN,), jnp.int32),
            pltpu.VMEM((2 * WIN,), jnp.float32),
            pltpu.VMEM((2 * WIN,), jnp.int32),
            pltpu.VMEM((2 * WIN * 16,), jnp.float32),
            pltpu.VMEM((ACCW,), jnp.float32),
            pltpu.VMEM((RED,), jnp.float32),
            pltpu.VMEM((RED,), jnp.float32),
            pltpu.VMEM_SHARED((16, ACCW), jnp.float32),
            pltpu.SemaphoreType.DMA((2,)),
        ],
    )
    return f(src, dst, elogp, edge_feats.reshape(E * 16), isgen, is42)


def _finalize_body(sums_ref, is42_ref, chg_ref, arm_ref, mxb_ref, nlp_ref,
                   gid_ref, cnt_ref, tl_ref, tr_ref, sl_acc, sr_acc):
    i = pl.program_id(0)

    def row(k):
        return jnp.sum(sums_ref[k:k + 1, :], axis=0)

    arom_sum = row(0) + row(4)
    bonds = row(1) + row(5)
    deg = row(2) + row(6)
    selp = row(3) + row(7)

    is42 = is42_ref[...] > 0.5
    charges = chg_ref[...]
    aromflag = arm_ref[...] > 0.5
    maxbonds = mxb_ref[...]
    nlp = nlp_ref[...]
    gid = gid_ref[...]
    counts = cnt_ref[...]

    has_arom = arom_sum > 0.0
    af = (has_arom != aromflag).astype(jnp.float32)
    valency = bonds - charges
    vf = (valency > maxbonds).astype(jnp.float32)
    bf = (is42 != (bonds == 0.0)).astype(jnp.float32)
    reward = -1.0 * af - 2.0 * vf - 3.0 * bf

    gi = lax.broadcasted_iota(jnp.int32, (NB, G), 1)
    rowpos = lax.broadcasted_iota(jnp.int32, (NB, G), 0)
    valid = rowpos + i * NB < N
    goh = gid[:, None] == gi
    cb = counts.reshape(1, G)
    nn = jnp.sum(jnp.where(goh, cb, 0.0), axis=1)
    reward = jnp.where((nn == 1.0) & is42, -4.0, reward)

    selp_mean = selp / jnp.maximum(deg, 1.0)
    loss = -(nlp + selp_mean) * reward

    gohv = goh & valid
    ploss = jnp.sum(jnp.where(gohv, loss[:, None], 0.0), axis=0)
    prew = jnp.sum(jnp.where(gohv, reward[:, None], 0.0), axis=0)

    @pl.when(i == 0)
    def _():
        sl_acc[...] = jnp.zeros_like(sl_acc)
        sr_acc[...] = jnp.zeros_like(sr_acc)

    sl_acc[...] += ploss
    sr_acc[...] += prew

    @pl.when(i == NGRID - 1)
    def _():
        c = jnp.maximum(counts, 1.0)
        tl_ref[...] = (jnp.sum(sl_acc[...] / c) / G).reshape(1, 1)
        tr_ref[...] = (jnp.sum(sr_acc[...] / c) / G).reshape(1, 1)


def _finalize(sums8, is42, chg, arm, mxb, nlp, graph_ids, counts):
    return pl.pallas_call(
        _finalize_body,
        grid=(NGRID,),
        in_specs=[
            pl.BlockSpec((8, NB), lambda i: (0, i)),
            pl.BlockSpec((NB,), lambda i: (i,)),
            pl.BlockSpec((NB,), lambda i: (i,)),
            pl.BlockSpec((NB,), lambda i: (i,)),
            pl.BlockSpec((NB,), lambda i: (i,)),
            pl.BlockSpec((NB,), lambda i: (i,)),
            pl.BlockSpec((NB,), lambda i: (i,)),
            pl.BlockSpec((G,), lambda i: (0,)),
        ],
        out_specs=[
            pl.BlockSpec((1, 1), lambda i: (0, 0)),
            pl.BlockSpec((1, 1), lambda i: (0, 0)),
        ],
        out_shape=[jax.ShapeDtypeStruct((1, 1), jnp.float32) for _ in range(2)],
        scratch_shapes=[
            pltpu.VMEM((G,), jnp.float32),
            pltpu.VMEM((G,), jnp.float32),
        ],
    )(sums8, is42, chg, arm, mxb, nlp, graph_ids, counts)


@jax.jit
def kernel(x, edge_feats, node_logprobs, edge_logprobs, max_bonds_table,
           edge_index, isgen, graph_ids):
    tab_pad = jnp.zeros((D,), jnp.float32).at[:43].set(max_bonds_table)
    is42, chg, arm, mxb, counts = _node_feats(x, graph_ids, tab_pad)
    src = edge_index[0]
    dst = edge_index[1]
    parts = _sc_edge_aggregate(src, dst, edge_logprobs, edge_feats, isgen,
                               is42)
    sums = parts.reshape(2, 4, NP)[:, :, :N].reshape(8, N)
    tl, tr = _finalize(sums, is42, chg, arm, mxb, node_logprobs, graph_ids,
                       counts)
    return (tl[0, 0], tr[0, 0])


# speedup vs baseline: 25.5713x; 1.0890x over previous
import functools

import jax
import jax.numpy as jnp
from jax import lax
from jax.experimental import pallas as pl
from jax.experimental.pallas import tpu as pltpu
from jax.experimental.pallas import tpu_sc as plsc

N = 10000
E = 320000
D = 128
G = 256

NP = 10240
NW = 32
EPW = E // NW
WIN = 400
NVREG = WIN // 16
NWIN = EPW // WIN
ACCW = 4 * NP
RED = ACCW // 16

NB = 2048
NGRID = (N + NB - 1) // NB
EB = 4096
EGRID = (E + EB - 1) // EB


def _node_feats_body(x_ref, gid_ref, tab_ref, is42_ref, chg_ref, arm_ref,
                     mxb_ref, cnt_ref):
    i = pl.program_id(0)
    xb = x_ref[...]
    lane = lax.broadcasted_iota(jnp.int32, (NB, D), 1)

    neg = jnp.float32(-3.4e38)
    m_at = lane < 43
    mx = jnp.max(jnp.where(m_at, xb, neg), axis=1, keepdims=True)
    cand = jnp.where((xb == mx) & m_at, lane, D)
    atoms = jnp.min(cand, axis=1)
    is42 = (atoms == 42).astype(jnp.float32)
    is42_ref[...] = is42

    m_ch = (lane >= 43) & (lane < 50)
    mx2 = jnp.max(jnp.where(m_ch, xb, neg), axis=1, keepdims=True)
    cand2 = jnp.where((xb == mx2) & m_ch, lane, D)
    chg_ref[...] = (jnp.min(cand2, axis=1) - 46).astype(jnp.float32)

    arm_ref[...] = (xb[:, 127] > xb[:, 126]).astype(jnp.float32)

    tab = tab_ref[...].reshape(1, D)
    onehot = lane == atoms[:, None]
    mxb_ref[...] = jnp.sum(jnp.where(onehot, tab, 0.0), axis=1)

    gid = gid_ref[...]
    gi = lax.broadcasted_iota(jnp.int32, (NB, G), 1)
    rowpos = lax.broadcasted_iota(jnp.int32, (NB, G), 0)
    valid = rowpos + i * NB < N
    goh = (gid[:, None] == gi) & valid
    part = jnp.sum(goh.astype(jnp.float32), axis=0)

    @pl.when(i == 0)
    def _():
        cnt_ref[...] = jnp.zeros_like(cnt_ref)

    cnt_ref[...] += part


def _node_feats(x, graph_ids, tab_pad):
    out = [jax.ShapeDtypeStruct((N,), jnp.float32) for _ in range(4)]
    out.append(jax.ShapeDtypeStruct((G,), jnp.float32))
    return pl.pallas_call(
        _node_feats_body,
        grid=(NGRID,),
        in_specs=[
            pl.BlockSpec((NB, D), lambda i: (i, 0)),
            pl.BlockSpec((NB,), lambda i: (i,)),
            pl.BlockSpec((D,), lambda i: (0,)),
        ],
        out_specs=[
            pl.BlockSpec((NB,), lambda i: (i,)),
            pl.BlockSpec((NB,), lambda i: (i,)),
            pl.BlockSpec((NB,), lambda i: (i,)),
            pl.BlockSpec((NB,), lambda i: (i,)),
            pl.BlockSpec((G,), lambda i: (0,)),
        ],
        out_shape=out,
    )(x, graph_ids, tab_pad)


def _sc_body(src_hbm, dst_hbm, lp_hbm, ef_hbm, ig_hbm, is42_hbm, out_hbm,
             is42_v, srcb, dstb, lpb, igb, efb, acc, res, tmp, spmem, sems):
    core = lax.axis_index("c")
    sub = lax.axis_index("s")
    wid = sub * 2 + core
    ebase = wid * EPW

    def zero_body(j, _):
        acc[pl.ds(j * 16, 16)] = jnp.zeros((16,), jnp.float32)
        return 0

    lax.fori_loop(0, ACCW // 16, zero_body, 0)

    pltpu.sync_copy(is42_hbm, is42_v.at[pl.ds(0, N)])

    ones16 = jnp.ones((16,), jnp.float32)

    def issue(w, bi):
        off = ebase + w * WIN
        sl = pl.ds(bi * WIN, WIN)
        sem = sems.at[bi]
        return [
            pltpu.async_copy(src_hbm.at[pl.ds(off, WIN)], srcb.at[sl], sem),
            pltpu.async_copy(dst_hbm.at[pl.ds(off, WIN)], dstb.at[sl], sem),
            pltpu.async_copy(lp_hbm.at[pl.ds(off, WIN)], lpb.at[sl], sem),
            pltpu.async_copy(ig_hbm.at[pl.ds(off, WIN)], igb.at[sl], sem),
            pltpu.async_copy(ef_hbm.at[pl.ds(off * 16, WIN * 16)],
                             efb.at[pl.ds(bi * WIN * 16, WIN * 16)], sem),
        ]

    pending = {0: issue(0, 0)}

    for w in range(NWIN):
        bi = w % 2
        if w + 1 < NWIN:
            pending[(w + 1) % 2] = issue(w + 1, (w + 1) % 2)
        for h in pending[bi]:
            h.wait()
        ob = bi * WIN

        def vreg_body(v, _):
            o = ob + v * 16
            s = srcb[pl.ds(o, 16)]
            d = dstb[pl.ds(o, 16)]
            s42 = plsc.load_gather(is42_v, [s])
            d42 = plsc.load_gather(is42_v, [d])
            base16 = (lax.iota(jnp.int32, 16) + o) * 16
            m = plsc.load_gather(efb, [base16])
            b = jnp.zeros((16,), jnp.float32)
            for j in range(1, 5):
                cj = plsc.load_gather(efb, [base16 + j])
                gt = cj > m
                b = jnp.where(gt, jnp.float32(j), b)
                m = jnp.where(gt, cj, m)
            b4 = (b == 4.0).astype(jnp.float32)
            b2 = jnp.where(b == 4.0, 1.0, b)
            b2 = jnp.where(igb[pl.ds(o, 16)] == -1, 0.0, b2)
            lp = lpb[pl.ds(o, 16)]
            arom = b4 * (1.0 - s42) * (1.0 - d42)
            bond = b2 * (1.0 - s42)
            plsc.addupdate_scatter(acc, [d], arom)
            plsc.addupdate_scatter(acc, [d + NP], bond)
            plsc.addupdate_scatter(acc, [d + 2 * NP], ones16)
            plsc.addupdate_scatter(acc, [d + 3 * NP], lp)
            return 0

        lax.fori_loop(0, NVREG, vreg_body, 0)

    pltpu.sync_copy(acc, spmem.at[sub])
    plsc.subcore_barrier()

    rbase = sub * RED
    pltpu.sync_copy(spmem.at[0, pl.ds(rbase, RED)], res)

    def red_body(i, _):
        pltpu.sync_copy(spmem.at[i, pl.ds(rbase, RED)], tmp)

        def add_body(j, _):
            sl = pl.ds(j * 16, 16)
            res[sl] += tmp[sl]
            return 0

        lax.fori_loop(0, RED // 16, add_body, 0)
        return 0

    lax.fori_loop(1, 16, red_body, 0)

    pltpu.sync_copy(res, out_hbm.at[core, pl.ds(rbase, RED)])


def _sc_edge_aggregate(src, dst, elogp, edge_feats, isgen, is42):
    mesh = plsc.VectorSubcoreMesh(core_axis_name="c", subcore_axis_name="s")
    f = pl.kernel(
        _sc_body,
        mesh=mesh,
        compiler_params=pltpu.CompilerParams(needs_layout_passes=False),
        out_type=jax.ShapeDtypeStruct((2, ACCW), jnp.float32),
        scratch_types=[
            pltpu.VMEM((NP,), jnp.float32),
            pltpu.VMEM((2 * WIN,), jnp.int32),
            pltpu.VMEM((2 * WIN,), jnp.int32),
            pltpu.VMEM((2 * WIN,), jnp.float32),
            pltpu.VMEM((2 * WIN,), jnp.int32),
            pltpu.VMEM((2 * WIN * 16,), jnp.float32),
            pltpu.VMEM((ACCW,), jnp.float32),
            pltpu.VMEM((RED,), jnp.float32),
            pltpu.VMEM((RED,), jnp.float32),
            pltpu.VMEM_SHARED((16, ACCW), jnp.float32),
            pltpu.SemaphoreType.DMA((2,)),
        ],
    )
    return f(src, dst, elogp, edge_feats.reshape(E * 16), isgen, is42)


def _finalize_body(sums_ref, is42_ref, chg_ref, arm_ref, mxb_ref, nlp_ref,
                   gid_ref, cnt_ref, tl_ref, tr_ref, sl_acc, sr_acc):
    i = pl.program_id(0)

    def row(k):
        return jnp.sum(sums_ref[k:k + 1, :], axis=0)

    arom_sum = row(0) + row(4)
    bonds = row(1) + row(5)
    deg = row(2) + row(6)
    selp = row(3) + row(7)

    is42 = is42_ref[...] > 0.5
    charges = chg_ref[...]
    aromflag = arm_ref[...] > 0.5
    maxbonds = mxb_ref[...]
    nlp = nlp_ref[...]
    gid = gid_ref[...]
    counts = cnt_ref[...]

    has_arom = arom_sum > 0.0
    af = (has_arom != aromflag).astype(jnp.float32)
    valency = bonds - charges
    vf = (valency > maxbonds).astype(jnp.float32)
    bf = (is42 != (bonds == 0.0)).astype(jnp.float32)
    reward = -1.0 * af - 2.0 * vf - 3.0 * bf

    gi = lax.broadcasted_iota(jnp.int32, (NB, G), 1)
    rowpos = lax.broadcasted_iota(jnp.int32, (NB, G), 0)
    valid = rowpos + i * NB < N
    goh = gid[:, None] == gi
    cb = counts.reshape(1, G)
    nn = jnp.sum(jnp.where(goh, cb, 0.0), axis=1)
    reward = jnp.where((nn == 1.0) & is42, -4.0, reward)

    selp_mean = selp / jnp.maximum(deg, 1.0)
    loss = -(nlp + selp_mean) * reward

    gohv = goh & valid
    ploss = jnp.sum(jnp.where(gohv, loss[:, None], 0.0), axis=0)
    prew = jnp.sum(jnp.where(gohv, reward[:, None], 0.0), axis=0)

    @pl.when(i == 0)
    def _():
        sl_acc[...] = jnp.zeros_like(sl_acc)
        sr_acc[...] = jnp.zeros_like(sr_acc)

    sl_acc[...] += ploss
    sr_acc[...] += prew

    @pl.when(i == NGRID - 1)
    def _():
        c = jnp.maximum(counts, 1.0)
        tl_ref[...] = (jnp.sum(sl_acc[...] / c) / G).reshape(1, 1)
        tr_ref[...] = (jnp.sum(sr_acc[...] / c) / G).reshape(1, 1)


def _finalize(sums8, is42, chg, arm, mxb, nlp, graph_ids, counts):
    return pl.pallas_call(
        _finalize_body,
        grid=(NGRID,),
        in_specs=[
            pl.BlockSpec((8, NB), lambda i: (0, i)),
            pl.BlockSpec((NB,), lambda i: (i,)),
            pl.BlockSpec((NB,), lambda i: (i,)),
            pl.BlockSpec((NB,), lambda i: (i,)),
            pl.BlockSpec((NB,), lambda i: (i,)),
            pl.BlockSpec((NB,), lambda i: (i,)),
            pl.BlockSpec((NB,), lambda i: (i,)),
            pl.BlockSpec((G,), lambda i: (0,)),
        ],
        out_specs=[
            pl.BlockSpec((1, 1), lambda i: (0, 0)),
            pl.BlockSpec((1, 1), lambda i: (0, 0)),
        ],
        out_shape=[jax.ShapeDtypeStruct((1, 1), jnp.float32) for _ in range(2)],
        scratch_shapes=[
            pltpu.VMEM((G,), jnp.float32),
            pltpu.VMEM((G,), jnp.float32),
        ],
    )(sums8, is42, chg, arm, mxb, nlp, graph_ids, counts)


@jax.jit
def kernel(x, edge_feats, node_logprobs, edge_logprobs, max_bonds_table,
           edge_index, isgen, graph_ids):
    tab_pad = jnp.zeros((D,), jnp.float32).at[:43].set(max_bonds_table)
    is42, chg, arm, mxb, counts = _node_feats(x, graph_ids, tab_pad)
    src = edge_index[0]
    dst = edge_index[1]
    parts = _sc_edge_aggregate(src, dst, edge_logprobs, edge_feats, isgen,
                               is42)
    sums = parts.reshape(2, 4, NP)[:, :, :N].reshape(8, N)
    tl, tr = _finalize(sums, is42, chg, arm, mxb, node_logprobs, graph_ids,
                       counts)
    return (tl[0, 0], tr[0, 0])


# speedup vs baseline: 25.6769x; 1.0041x over previous
import functools

import jax
import jax.numpy as jnp
from jax import lax
from jax.experimental import pallas as pl
from jax.experimental.pallas import tpu as pltpu
from jax.experimental.pallas import tpu_sc as plsc

N = 10000
E = 320000
D = 128
G = 256

NP = 10240
NW = 32
EPW = E // NW
WIN = 400
NVREG = WIN // 16
NWIN = EPW // WIN
ACCW = 4 * NP
RED = ACCW // 16

NB = 2048
NGRID = (N + NB - 1) // NB
EB = 4096
EGRID = (E + EB - 1) // EB


def _node_feats_body(x_ref, gid_ref, tab_ref, is42_ref, chg_ref, arm_ref,
                     mxb_ref, cnt_ref):
    i = pl.program_id(0)
    xb = x_ref[...]
    lane = lax.broadcasted_iota(jnp.int32, (NB, D), 1)

    neg = jnp.float32(-3.4e38)
    m_at = lane < 43
    mx = jnp.max(jnp.where(m_at, xb, neg), axis=1, keepdims=True)
    cand = jnp.where((xb == mx) & m_at, lane, D)
    atoms = jnp.min(cand, axis=1)
    is42 = (atoms == 42).astype(jnp.float32)
    is42_ref[...] = is42

    m_ch = (lane >= 43) & (lane < 50)
    mx2 = jnp.max(jnp.where(m_ch, xb, neg), axis=1, keepdims=True)
    cand2 = jnp.where((xb == mx2) & m_ch, lane, D)
    chg_ref[...] = (jnp.min(cand2, axis=1) - 46).astype(jnp.float32)

    arm_ref[...] = (xb[:, 127] > xb[:, 126]).astype(jnp.float32)

    tab = tab_ref[...].reshape(1, D)
    onehot = lane == atoms[:, None]
    mxb_ref[...] = jnp.sum(jnp.where(onehot, tab, 0.0), axis=1)

    gid = gid_ref[...]
    gi = lax.broadcasted_iota(jnp.int32, (NB, G), 1)
    rowpos = lax.broadcasted_iota(jnp.int32, (NB, G), 0)
    valid = rowpos + i * NB < N
    goh = (gid[:, None] == gi) & valid
    part = jnp.sum(goh.astype(jnp.float32), axis=0)

    @pl.when(i == 0)
    def _():
        cnt_ref[...] = jnp.zeros_like(cnt_ref)

    cnt_ref[...] += part


def _node_feats(x, graph_ids, tab_pad):
    out = [jax.ShapeDtypeStruct((N,), jnp.float32) for _ in range(4)]
    out.append(jax.ShapeDtypeStruct((G,), jnp.float32))
    return pl.pallas_call(
        _node_feats_body,
        grid=(NGRID,),
        in_specs=[
            pl.BlockSpec((NB, D), lambda i: (i, 0)),
            pl.BlockSpec((NB,), lambda i: (i,)),
            pl.BlockSpec((D,), lambda i: (0,)),
        ],
        out_specs=[
            pl.BlockSpec((NB,), lambda i: (i,)),
            pl.BlockSpec((NB,), lambda i: (i,)),
            pl.BlockSpec((NB,), lambda i: (i,)),
            pl.BlockSpec((NB,), lambda i: (i,)),
            pl.BlockSpec((G,), lambda i: (0,)),
        ],
        out_shape=out,
    )(x, graph_ids, tab_pad)


def _sc_body(src_hbm, dst_hbm, lp_hbm, ef_hbm, ig_hbm, is42_hbm, out_hbm,
             is42_v, srcb, dstb, lpb, igb, efb, acc, res, tmp, spmem, sems):
    core = lax.axis_index("c")
    sub = lax.axis_index("s")
    wid = sub * 2 + core
    ebase = wid * EPW

    def zero_body(j, _):
        acc[pl.ds(j * 16, 16)] = jnp.zeros((16,), jnp.float32)
        return 0

    lax.fori_loop(0, ACCW // 16, zero_body, 0)

    pltpu.sync_copy(is42_hbm, is42_v.at[pl.ds(0, N)])

    ones16 = jnp.ones((16,), jnp.float32)

    def issue(w, bi):
        off = ebase + w * WIN
        sl = pl.ds(bi * WIN, WIN)
        sem = sems.at[bi]
        return [
            pltpu.async_copy(src_hbm.at[pl.ds(off, WIN)], srcb.at[sl], sem),
            pltpu.async_copy(dst_hbm.at[pl.ds(off, WIN)], dstb.at[sl], sem),
            pltpu.async_copy(lp_hbm.at[pl.ds(off, WIN)], lpb.at[sl], sem),
            pltpu.async_copy(ig_hbm.at[pl.ds(off, WIN)], igb.at[sl], sem),
            pltpu.async_copy(ef_hbm.at[pl.ds(off * 16, WIN * 16)],
                             efb.at[pl.ds(bi * WIN * 16, WIN * 16)], sem),
        ]

    pending = {0: issue(0, 0)}

    for w in range(NWIN):
        bi = w % 2
        if w + 1 < NWIN:
            pending[(w + 1) % 2] = issue(w + 1, (w + 1) % 2)
        for h in pending[bi]:
            h.wait()
        ob = bi * WIN

        @plsc.parallel_loop(0, NVREG, unroll=2)
        def vreg_body(v):
            o = ob + v * 16
            s = srcb[pl.ds(o, 16)]
            d = dstb[pl.ds(o, 16)]
            s42 = plsc.load_gather(is42_v, [s])
            d42 = plsc.load_gather(is42_v, [d])
            base16 = (lax.iota(jnp.int32, 16) + o) * 16
            m = plsc.load_gather(efb, [base16])
            b = jnp.zeros((16,), jnp.float32)
            for j in range(1, 5):
                cj = plsc.load_gather(efb, [base16 + j])
                gt = cj > m
                b = jnp.where(gt, jnp.float32(j), b)
                m = jnp.where(gt, cj, m)
            b4 = (b == 4.0).astype(jnp.float32)
            b2 = jnp.where(b == 4.0, 1.0, b)
            b2 = jnp.where(igb[pl.ds(o, 16)] == -1, 0.0, b2)
            lp = lpb[pl.ds(o, 16)]
            arom = b4 * (1.0 - s42) * (1.0 - d42)
            bond = b2 * (1.0 - s42)
            plsc.addupdate_scatter(acc, [d], arom)
            plsc.addupdate_scatter(acc, [d + NP], bond)
            plsc.addupdate_scatter(acc, [d + 2 * NP], ones16)
            plsc.addupdate_scatter(acc, [d + 3 * NP], lp)

    pltpu.sync_copy(acc, spmem.at[sub])
    plsc.subcore_barrier()

    rbase = sub * RED
    pltpu.sync_copy(spmem.at[0, pl.ds(rbase, RED)], res)

    def red_body(i, _):
        pltpu.sync_copy(spmem.at[i, pl.ds(rbase, RED)], tmp)

        def add_body(j, _):
            sl = pl.ds(j * 16, 16)
            res[sl] += tmp[sl]
            return 0

        lax.fori_loop(0, RED // 16, add_body, 0)
        return 0

    lax.fori_loop(1, 16, red_body, 0)

    pltpu.sync_copy(res, out_hbm.at[core, pl.ds(rbase, RED)])


def _sc_edge_aggregate(src, dst, elogp, edge_feats, isgen, is42):
    mesh = plsc.VectorSubcoreMesh(core_axis_name="c", subcore_axis_name="s")
    f = pl.kernel(
        _sc_body,
        mesh=mesh,
        compiler_params=pltpu.CompilerParams(needs_layout_passes=False),
        out_type=jax.ShapeDtypeStruct((2, ACCW), jnp.float32),
        scratch_types=[
            pltpu.VMEM((NP,), jnp.float32),
            pltpu.VMEM((2 * WIN,), jnp.int32),
            pltpu.VMEM((2 * WIN,), jnp.int32),
            pltpu.VMEM((2 * WIN,), jnp.float32),
            pltpu.VMEM((2 * WIN,), jnp.int32),
            pltpu.VMEM((2 * WIN * 16,), jnp.float32),
            pltpu.VMEM((ACCW,), jnp.float32),
            pltpu.VMEM((RED,), jnp.float32),
            pltpu.VMEM((RED,), jnp.float32),
            pltpu.VMEM_SHARED((16, ACCW), jnp.float32),
            pltpu.SemaphoreType.DMA((2,)),
        ],
    )
    return f(src, dst, elogp, edge_feats.reshape(E * 16), isgen, is42)


def _finalize_body(sums_ref, is42_ref, chg_ref, arm_ref, mxb_ref, nlp_ref,
                   gid_ref, cnt_ref, tl_ref, tr_ref, sl_acc, sr_acc):
    i = pl.program_id(0)

    def row(k):
        return jnp.sum(sums_ref[k:k + 1, :], axis=0)

    arom_sum = row(0) + row(4)
    bonds = row(1) + row(5)
    deg = row(2) + row(6)
    selp = row(3) + row(7)

    is42 = is42_ref[...] > 0.5
    charges = chg_ref[...]
    aromflag = arm_ref[...] > 0.5
    maxbonds = mxb_ref[...]
    nlp = nlp_ref[...]
    gid = gid_ref[...]
    counts = cnt_ref[...]

    has_arom = arom_sum > 0.0
    af = (has_arom != aromflag).astype(jnp.float32)
    valency = bonds - charges
    vf = (valency > maxbonds).astype(jnp.float32)
    bf = (is42 != (bonds == 0.0)).astype(jnp.float32)
    reward = -1.0 * af - 2.0 * vf - 3.0 * bf

    gi = lax.broadcasted_iota(jnp.int32, (NB, G), 1)
    rowpos = lax.broadcasted_iota(jnp.int32, (NB, G), 0)
    valid = rowpos + i * NB < N
    goh = gid[:, None] == gi
    cb = counts.reshape(1, G)
    nn = jnp.sum(jnp.where(goh, cb, 0.0), axis=1)
    reward = jnp.where((nn == 1.0) & is42, -4.0, reward)

    selp_mean = selp / jnp.maximum(deg, 1.0)
    loss = -(nlp + selp_mean) * reward

    gohv = goh & valid
    ploss = jnp.sum(jnp.where(gohv, loss[:, None], 0.0), axis=0)
    prew = jnp.sum(jnp.where(gohv, reward[:, None], 0.0), axis=0)

    @pl.when(i == 0)
    def _():
        sl_acc[...] = jnp.zeros_like(sl_acc)
        sr_acc[...] = jnp.zeros_like(sr_acc)

    sl_acc[...] += ploss
    sr_acc[...] += prew

    @pl.when(i == NGRID - 1)
    def _():
        c = jnp.maximum(counts, 1.0)
        tl_ref[...] = (jnp.sum(sl_acc[...] / c) / G).reshape(1, 1)
        tr_ref[...] = (jnp.sum(sr_acc[...] / c) / G).reshape(1, 1)


def _finalize(sums8, is42, chg, arm, mxb, nlp, graph_ids, counts):
    return pl.pallas_call(
        _finalize_body,
        grid=(NGRID,),
        in_specs=[
            pl.BlockSpec((8, NB), lambda i: (0, i)),
            pl.BlockSpec((NB,), lambda i: (i,)),
            pl.BlockSpec((NB,), lambda i: (i,)),
            pl.BlockSpec((NB,), lambda i: (i,)),
            pl.BlockSpec((NB,), lambda i: (i,)),
            pl.BlockSpec((NB,), lambda i: (i,)),
            pl.BlockSpec((NB,), lambda i: (i,)),
            pl.BlockSpec((G,), lambda i: (0,)),
        ],
        out_specs=[
            pl.BlockSpec((1, 1), lambda i: (0, 0)),
            pl.BlockSpec((1, 1), lambda i: (0, 0)),
        ],
        out_shape=[jax.ShapeDtypeStruct((1, 1), jnp.float32) for _ in range(2)],
        scratch_shapes=[
            pltpu.VMEM((G,), jnp.float32),
            pltpu.VMEM((G,), jnp.float32),
        ],
    )(sums8, is42, chg, arm, mxb, nlp, graph_ids, counts)


@jax.jit
def kernel(x, edge_feats, node_logprobs, edge_logprobs, max_bonds_table,
           edge_index, isgen, graph_ids):
    tab_pad = jnp.zeros((D,), jnp.float32).at[:43].set(max_bonds_table)
    is42, chg, arm, mxb, counts = _node_feats(x, graph_ids, tab_pad)
    src = edge_index[0]
    dst = edge_index[1]
    parts = _sc_edge_aggregate(src, dst, edge_logprobs, edge_feats, isgen,
                               is42)
    sums = parts.reshape(2, 4, NP)[:, :, :N].reshape(8, N)
    tl, tr = _finalize(sums, is42, chg, arm, mxb, node_logprobs, graph_ids,
                       counts)
    return (tl[0, 0], tr[0, 0])


# speedup vs baseline: 26.4730x; 1.0310x over previous
import functools

import jax
import jax.numpy as jnp
from jax import lax
from jax.experimental import pallas as pl
from jax.experimental.pallas import tpu as pltpu
from jax.experimental.pallas import tpu_sc as plsc

N = 10000
E = 320000
D = 128
G = 256

NP = 10240
NW = 32
EPW = E // NW
WIN = 400
NVREG = WIN // 16
NWIN = EPW // WIN
ACCW = 4 * NP
RED = ACCW // 16

NB = 2048
NGRID = (N + NB - 1) // NB
EB = 4096
EGRID = (E + EB - 1) // EB


def _node_feats_body(x_ref, gid_ref, tab_ref, is42_ref, chg_ref, arm_ref,
                     mxb_ref, cnt_ref):
    i = pl.program_id(0)
    xb = x_ref[...]
    lane = lax.broadcasted_iota(jnp.int32, (NB, D), 1)

    neg = jnp.float32(-3.4e38)
    m_at = lane < 43
    mx = jnp.max(jnp.where(m_at, xb, neg), axis=1, keepdims=True)
    cand = jnp.where((xb == mx) & m_at, lane, D)
    atoms = jnp.min(cand, axis=1)
    is42 = (atoms == 42).astype(jnp.float32)
    is42_ref[...] = is42

    m_ch = (lane >= 43) & (lane < 50)
    mx2 = jnp.max(jnp.where(m_ch, xb, neg), axis=1, keepdims=True)
    cand2 = jnp.where((xb == mx2) & m_ch, lane, D)
    chg_ref[...] = (jnp.min(cand2, axis=1) - 46).astype(jnp.float32)

    arm_ref[...] = (xb[:, 127] > xb[:, 126]).astype(jnp.float32)

    tab = tab_ref[...].reshape(1, D)
    onehot = lane == atoms[:, None]
    mxb_ref[...] = jnp.sum(jnp.where(onehot, tab, 0.0), axis=1)

    gid = gid_ref[...]
    gi = lax.broadcasted_iota(jnp.int32, (NB, G), 1)
    rowpos = lax.broadcasted_iota(jnp.int32, (NB, G), 0)
    valid = rowpos + i * NB < N
    goh = (gid[:, None] == gi) & valid
    part = jnp.sum(goh.astype(jnp.float32), axis=0)

    @pl.when(i == 0)
    def _():
        cnt_ref[...] = jnp.zeros_like(cnt_ref)

    cnt_ref[...] += part


def _node_feats(x, graph_ids, tab_pad):
    out = [jax.ShapeDtypeStruct((N,), jnp.float32) for _ in range(4)]
    out.append(jax.ShapeDtypeStruct((G,), jnp.float32))
    return pl.pallas_call(
        _node_feats_body,
        grid=(NGRID,),
        in_specs=[
            pl.BlockSpec((NB, D), lambda i: (i, 0)),
            pl.BlockSpec((NB,), lambda i: (i,)),
            pl.BlockSpec((D,), lambda i: (0,)),
        ],
        out_specs=[
            pl.BlockSpec((NB,), lambda i: (i,)),
            pl.BlockSpec((NB,), lambda i: (i,)),
            pl.BlockSpec((NB,), lambda i: (i,)),
            pl.BlockSpec((NB,), lambda i: (i,)),
            pl.BlockSpec((G,), lambda i: (0,)),
        ],
        out_shape=out,
    )(x, graph_ids, tab_pad)


def _sc_body(ei_hbm, lp_hbm, ef_hbm, ig_hbm, is42_hbm, out_hbm,
             is42_v, srcb, dstb, lpb, igb, efb, acc, res, tmp, spmem, sems):
    core = lax.axis_index("c")
    sub = lax.axis_index("s")
    wid = sub * 2 + core
    ebase = wid * EPW

    def zero_body(j, _):
        acc[pl.ds(j * 16, 16)] = jnp.zeros((16,), jnp.float32)
        return 0

    lax.fori_loop(0, ACCW // 16, zero_body, 0)

    pltpu.sync_copy(is42_hbm, is42_v.at[pl.ds(0, N)])

    ones16 = jnp.ones((16,), jnp.float32)

    def issue(w, bi):
        off = ebase + w * WIN
        sl = pl.ds(bi * WIN, WIN)
        sem = sems.at[bi]
        return [
            pltpu.async_copy(ei_hbm.at[0, pl.ds(off, WIN)], srcb.at[sl], sem),
            pltpu.async_copy(ei_hbm.at[1, pl.ds(off, WIN)], dstb.at[sl], sem),
            pltpu.async_copy(lp_hbm.at[pl.ds(off, WIN)], lpb.at[sl], sem),
            pltpu.async_copy(ig_hbm.at[pl.ds(off, WIN)], igb.at[sl], sem),
            pltpu.async_copy(ef_hbm.at[pl.ds(off, WIN), :],
                             efb.at[pl.ds(bi * WIN, WIN), :], sem),
        ]

    pending = {0: issue(0, 0)}

    for w in range(NWIN):
        bi = w % 2
        if w + 1 < NWIN:
            pending[(w + 1) % 2] = issue(w + 1, (w + 1) % 2)
        for h in pending[bi]:
            h.wait()
        ob = bi * WIN

        @plsc.parallel_loop(0, NVREG, unroll=2)
        def vreg_body(v):
            o = ob + v * 16
            s = srcb[pl.ds(o, 16)]
            d = dstb[pl.ds(o, 16)]
            s42 = plsc.load_gather(is42_v, [s])
            d42 = plsc.load_gather(is42_v, [d])
            rows = lax.iota(jnp.int32, 16) + o
            m = plsc.load_gather(efb, [rows, jnp.zeros((16,), jnp.int32)])
            b = jnp.zeros((16,), jnp.float32)
            for j in range(1, 5):
                cj = plsc.load_gather(efb, [rows, jnp.full((16,), j,
                                                           jnp.int32)])
                gt = cj > m
                b = jnp.where(gt, jnp.float32(j), b)
                m = jnp.where(gt, cj, m)
            b4 = (b == 4.0).astype(jnp.float32)
            b2 = jnp.where(b == 4.0, 1.0, b)
            b2 = jnp.where(igb[pl.ds(o, 16)] == -1, 0.0, b2)
            lp = lpb[pl.ds(o, 16)]
            arom = b4 * (1.0 - s42) * (1.0 - d42)
            bond = b2 * (1.0 - s42)
            plsc.addupdate_scatter(acc, [d], arom)
            plsc.addupdate_scatter(acc, [d + NP], bond)
            plsc.addupdate_scatter(acc, [d + 2 * NP], ones16)
            plsc.addupdate_scatter(acc, [d + 3 * NP], lp)

    pltpu.sync_copy(acc, spmem.at[sub])
    plsc.subcore_barrier()

    rbase = sub * RED
    pltpu.sync_copy(spmem.at[0, pl.ds(rbase, RED)], res)

    def red_body(i, _):
        pltpu.sync_copy(spmem.at[i, pl.ds(rbase, RED)], tmp)

        def add_body(j, _):
            sl = pl.ds(j * 16, 16)
            res[sl] += tmp[sl]
            return 0

        lax.fori_loop(0, RED // 16, add_body, 0)
        return 0

    lax.fori_loop(1, 16, red_body, 0)

    pltpu.sync_copy(res, out_hbm.at[core, pl.ds(rbase, RED)])


def _sc_edge_aggregate(edge_index, elogp, edge_feats, isgen, is42):
    mesh = plsc.VectorSubcoreMesh(core_axis_name="c", subcore_axis_name="s")
    f = pl.kernel(
        _sc_body,
        mesh=mesh,
        compiler_params=pltpu.CompilerParams(needs_layout_passes=False,
                                             use_tc_tiling_on_sc=False),
        out_type=jax.ShapeDtypeStruct((2, ACCW), jnp.float32),
        scratch_types=[
            pltpu.VMEM((NP,), jnp.float32),
            pltpu.VMEM((2 * WIN,), jnp.int32),
            pltpu.VMEM((2 * WIN,), jnp.int32),
            pltpu.VMEM((2 * WIN,), jnp.float32),
            pltpu.VMEM((2 * WIN,), jnp.int32),
            pltpu.VMEM((2 * WIN, 16), jnp.float32),
            pltpu.VMEM((ACCW,), jnp.float32),
            pltpu.VMEM((RED,), jnp.float32),
            pltpu.VMEM((RED,), jnp.float32),
            pltpu.VMEM_SHARED((16, ACCW), jnp.float32),
            pltpu.SemaphoreType.DMA((2,)),
        ],
    )
    return f(edge_index, elogp, edge_feats, isgen, is42)


def _finalize_body(sums_ref, is42_ref, chg_ref, arm_ref, mxb_ref, nlp_ref,
                   gid_ref, cnt_ref, tl_ref, tr_ref, sl_acc, sr_acc):
    i = pl.program_id(0)

    def row(k):
        return jnp.sum(sums_ref[k:k + 1, :], axis=0)

    arom_sum = row(0) + row(4)
    bonds = row(1) + row(5)
    deg = row(2) + row(6)
    selp = row(3) + row(7)

    is42 = is42_ref[...] > 0.5
    charges = chg_ref[...]
    aromflag = arm_ref[...] > 0.5
    maxbonds = mxb_ref[...]
    nlp = nlp_ref[...]
    gid = gid_ref[...]
    counts = cnt_ref[...]

    has_arom = arom_sum > 0.0
    af = (has_arom != aromflag).astype(jnp.float32)
    valency = bonds - charges
    vf = (valency > maxbonds).astype(jnp.float32)
    bf = (is42 != (bonds == 0.0)).astype(jnp.float32)
    reward = -1.0 * af - 2.0 * vf - 3.0 * bf

    gi = lax.broadcasted_iota(jnp.int32, (NB, G), 1)
    rowpos = lax.broadcasted_iota(jnp.int32, (NB, G), 0)
    valid = rowpos + i * NB < N
    goh = gid[:, None] == gi
    cb = counts.reshape(1, G)
    nn = jnp.sum(jnp.where(goh, cb, 0.0), axis=1)
    reward = jnp.where((nn == 1.0) & is42, -4.0, reward)

    selp_mean = selp / jnp.maximum(deg, 1.0)
    loss = -(nlp + selp_mean) * reward

    gohv = goh & valid
    ploss = jnp.sum(jnp.where(gohv, loss[:, None], 0.0), axis=0)
    prew = jnp.sum(jnp.where(gohv, reward[:, None], 0.0), axis=0)

    @pl.when(i == 0)
    def _():
        sl_acc[...] = jnp.zeros_like(sl_acc)
        sr_acc[...] = jnp.zeros_like(sr_acc)

    sl_acc[...] += ploss
    sr_acc[...] += prew

    @pl.when(i == NGRID - 1)
    def _():
        c = jnp.maximum(counts, 1.0)
        tl_ref[...] = (jnp.sum(sl_acc[...] / c) / G).reshape(1, 1)
        tr_ref[...] = (jnp.sum(sr_acc[...] / c) / G).reshape(1, 1)


def _finalize(sums8, is42, chg, arm, mxb, nlp, graph_ids, counts):
    return pl.pallas_call(
        _finalize_body,
        grid=(NGRID,),
        in_specs=[
            pl.BlockSpec((8, NB), lambda i: (0, i)),
            pl.BlockSpec((NB,), lambda i: (i,)),
            pl.BlockSpec((NB,), lambda i: (i,)),
            pl.BlockSpec((NB,), lambda i: (i,)),
            pl.BlockSpec((NB,), lambda i: (i,)),
            pl.BlockSpec((NB,), lambda i: (i,)),
            pl.BlockSpec((NB,), lambda i: (i,)),
            pl.BlockSpec((G,), lambda i: (0,)),
        ],
        out_specs=[
            pl.BlockSpec((1, 1), lambda i: (0, 0)),
            pl.BlockSpec((1, 1), lambda i: (0, 0)),
        ],
        out_shape=[jax.ShapeDtypeStruct((1, 1), jnp.float32) for _ in range(2)],
        scratch_shapes=[
            pltpu.VMEM((G,), jnp.float32),
            pltpu.VMEM((G,), jnp.float32),
        ],
    )(sums8, is42, chg, arm, mxb, nlp, graph_ids, counts)


@jax.jit
def kernel(x, edge_feats, node_logprobs, edge_logprobs, max_bonds_table,
           edge_index, isgen, graph_ids):
    tab_pad = jnp.zeros((D,), jnp.float32).at[:43].set(max_bonds_table)
    is42, chg, arm, mxb, counts = _node_feats(x, graph_ids, tab_pad)
    parts = _sc_edge_aggregate(edge_index, edge_logprobs, edge_feats, isgen,
                               is42)
    sums = parts.reshape(2, 4, NP)[:, :, :N].reshape(8, N)
    tl, tr = _finalize(sums, is42, chg, arm, mxb, node_logprobs, graph_ids,
                       counts)
    return (tl[0, 0], tr[0, 0])


# speedup vs baseline: 26.5290x; 1.0021x over previous
import functools

import jax
import jax.numpy as jnp
from jax import lax
from jax.experimental import pallas as pl
from jax.experimental.pallas import tpu as pltpu
from jax.experimental.pallas import tpu_sc as plsc

N = 10000
E = 320000
D = 128
G = 256

NP = 10240
NW = 32
EPW = E // NW
WIN = 400
NVREG = WIN // 16
NWIN = EPW // WIN
ACCW = 4 * NP
RED = ACCW // 16

NB = 2048
NGRID = (N + NB - 1) // NB
EB = 4096
EGRID = (E + EB - 1) // EB


def _node_feats_body(x_ref, gid_ref, tab_ref, is42_ref, chg_ref, arm_ref,
                     mxb_ref, cnt_ref):
    i = pl.program_id(0)
    xb = x_ref[...]
    lane = lax.broadcasted_iota(jnp.int32, (NB, D), 1)

    neg = jnp.float32(-3.4e38)
    m_at = lane < 43
    mx = jnp.max(jnp.where(m_at, xb, neg), axis=1, keepdims=True)
    cand = jnp.where((xb == mx) & m_at, lane, D)
    atoms = jnp.min(cand, axis=1)
    is42 = (atoms == 42).astype(jnp.float32)
    is42_ref[...] = is42

    m_ch = (lane >= 43) & (lane < 50)
    mx2 = jnp.max(jnp.where(m_ch, xb, neg), axis=1, keepdims=True)
    cand2 = jnp.where((xb == mx2) & m_ch, lane, D)
    chg_ref[...] = (jnp.min(cand2, axis=1) - 46).astype(jnp.float32)

    arm_ref[...] = (xb[:, 127] > xb[:, 126]).astype(jnp.float32)

    tab = tab_ref[...].reshape(1, D)
    onehot = lane == atoms[:, None]
    mxb_ref[...] = jnp.sum(jnp.where(onehot, tab, 0.0), axis=1)

    gid = gid_ref[...]
    gi = lax.broadcasted_iota(jnp.int32, (NB, G), 1)
    rowpos = lax.broadcasted_iota(jnp.int32, (NB, G), 0)
    valid = rowpos + i * NB < N
    goh = (gid[:, None] == gi) & valid
    part = jnp.sum(goh.astype(jnp.float32), axis=0)

    @pl.when(i == 0)
    def _():
        cnt_ref[...] = jnp.zeros_like(cnt_ref)

    cnt_ref[...] += part


def _node_feats(x, graph_ids, tab_pad):
    out = [jax.ShapeDtypeStruct((N,), jnp.float32) for _ in range(4)]
    out.append(jax.ShapeDtypeStruct((G,), jnp.float32))
    return pl.pallas_call(
        _node_feats_body,
        grid=(NGRID,),
        in_specs=[
            pl.BlockSpec((NB, D), lambda i: (i, 0)),
            pl.BlockSpec((NB,), lambda i: (i,)),
            pl.BlockSpec((D,), lambda i: (0,)),
        ],
        out_specs=[
            pl.BlockSpec((NB,), lambda i: (i,)),
            pl.BlockSpec((NB,), lambda i: (i,)),
            pl.BlockSpec((NB,), lambda i: (i,)),
            pl.BlockSpec((NB,), lambda i: (i,)),
            pl.BlockSpec((G,), lambda i: (0,)),
        ],
        out_shape=out,
    )(x, graph_ids, tab_pad)


def _sc_body(ei_hbm, lp_hbm, ef_hbm, ig_hbm, is42_hbm, out_hbm,
             is42_v, srcb, dstb, lpb, igb, efb, acc, res, tmp, spmem, sems):
    core = lax.axis_index("c")
    sub = lax.axis_index("s")
    wid = sub * 2 + core
    ebase = wid * EPW

    def zero_body(j, _):
        acc[pl.ds(j * 16, 16)] = jnp.zeros((16,), jnp.float32)
        return 0

    lax.fori_loop(0, ACCW // 16, zero_body, 0)

    pltpu.sync_copy(is42_hbm, is42_v.at[pl.ds(0, N)])

    ones16 = jnp.ones((16,), jnp.float32)

    def issue(w, bi):
        off = ebase + w * WIN
        sl = pl.ds(bi * WIN, WIN)
        sem = sems.at[bi]
        return [
            pltpu.async_copy(ei_hbm.at[0, pl.ds(off, WIN)], srcb.at[sl], sem),
            pltpu.async_copy(ei_hbm.at[1, pl.ds(off, WIN)], dstb.at[sl], sem),
            pltpu.async_copy(lp_hbm.at[pl.ds(off, WIN)], lpb.at[sl], sem),
            pltpu.async_copy(ig_hbm.at[pl.ds(off, WIN)], igb.at[sl], sem),
            pltpu.async_copy(
                ef_hbm.at[pl.ds(off // 8, WIN // 8), :],
                efb.at[pl.ds(bi * (WIN // 8), WIN // 8), :], sem),
        ]

    pending = {0: issue(0, 0)}

    for w in range(NWIN):
        bi = w % 2
        if w + 1 < NWIN:
            pending[(w + 1) % 2] = issue(w + 1, (w + 1) % 2)
        for h in pending[bi]:
            h.wait()
        ob = bi * WIN

        @plsc.parallel_loop(0, NVREG, unroll=2)
        def vreg_body(v):
            o = ob + v * 16
            s = srcb[pl.ds(o, 16)]
            d = dstb[pl.ds(o, 16)]
            s42 = plsc.load_gather(is42_v, [s])
            d42 = plsc.load_gather(is42_v, [d])
            base16 = (lax.iota(jnp.int32, 16) + o) * 16
            er = lax.shift_right_logical(base16, 7)
            ec = jnp.bitwise_and(base16, 127)
            m = plsc.load_gather(efb, [er, ec])
            b = jnp.zeros((16,), jnp.float32)
            for j in range(1, 5):
                cj = plsc.load_gather(efb, [er, ec + j])
                gt = cj > m
                b = jnp.where(gt, jnp.float32(j), b)
                m = jnp.where(gt, cj, m)
            b4 = (b == 4.0).astype(jnp.float32)
            b2 = jnp.where(b == 4.0, 1.0, b)
            b2 = jnp.where(igb[pl.ds(o, 16)] == -1, 0.0, b2)
            lp = lpb[pl.ds(o, 16)]
            arom = b4 * (1.0 - s42) * (1.0 - d42)
            bond = b2 * (1.0 - s42)
            plsc.addupdate_scatter(acc, [d], arom)
            plsc.addupdate_scatter(acc, [d + NP], bond)
            plsc.addupdate_scatter(acc, [d + 2 * NP], ones16)
            plsc.addupdate_scatter(acc, [d + 3 * NP], lp)

    pltpu.sync_copy(acc, spmem.at[sub])
    plsc.subcore_barrier()

    rbase = sub * RED
    pltpu.sync_copy(spmem.at[0, pl.ds(rbase, RED)], res)

    def red_body(i, _):
        pltpu.sync_copy(spmem.at[i, pl.ds(rbase, RED)], tmp)

        def add_body(j, _):
            sl = pl.ds(j * 16, 16)
            res[sl] += tmp[sl]
            return 0

        lax.fori_loop(0, RED // 16, add_body, 0)
        return 0

    lax.fori_loop(1, 16, red_body, 0)

    pltpu.sync_copy(res, out_hbm.at[core, pl.ds(rbase, RED)])


def _sc_edge_aggregate(edge_index, elogp, edge_feats, isgen, is42):
    mesh = plsc.VectorSubcoreMesh(core_axis_name="c", subcore_axis_name="s")
    f = pl.kernel(
        _sc_body,
        mesh=mesh,
        compiler_params=pltpu.CompilerParams(needs_layout_passes=False,
                                             use_tc_tiling_on_sc=False),
        out_type=jax.ShapeDtypeStruct((2, ACCW), jnp.float32),
        scratch_types=[
            pltpu.VMEM((NP,), jnp.float32),
            pltpu.VMEM((2 * WIN,), jnp.int32),
            pltpu.VMEM((2 * WIN,), jnp.int32),
            pltpu.VMEM((2 * WIN,), jnp.float32),
            pltpu.VMEM((2 * WIN,), jnp.int32),
            pltpu.VMEM((2 * (WIN // 8), 128), jnp.float32),
            pltpu.VMEM((ACCW,), jnp.float32),
            pltpu.VMEM((RED,), jnp.float32),
            pltpu.VMEM((RED,), jnp.float32),
            pltpu.VMEM_SHARED((16, ACCW), jnp.float32),
            pltpu.SemaphoreType.DMA((2,)),
        ],
    )
    return f(edge_index, elogp, edge_feats.reshape(E // 8, 128), isgen, is42)


def _finalize_body(sums_ref, is42_ref, chg_ref, arm_ref, mxb_ref, nlp_ref,
                   gid_ref, cnt_ref, tl_ref, tr_ref, sl_acc, sr_acc):
    i = pl.program_id(0)

    def row(k):
        return jnp.sum(sums_ref[k:k + 1, :], axis=0)

    arom_sum = row(0) + row(4)
    bonds = row(1) + row(5)
    deg = row(2) + row(6)
    selp = row(3) + row(7)

    is42 = is42_ref[...] > 0.5
    charges = chg_ref[...]
    aromflag = arm_ref[...] > 0.5
    maxbonds = mxb_ref[...]
    nlp = nlp_ref[...]
    gid = gid_ref[...]
    counts = cnt_ref[...]

    has_arom = arom_sum > 0.0
    af = (has_arom != aromflag).astype(jnp.float32)
    valency = bonds - charges
    vf = (valency > maxbonds).astype(jnp.float32)
    bf = (is42 != (bonds == 0.0)).astype(jnp.float32)
    reward = -1.0 * af - 2.0 * vf - 3.0 * bf

    gi = lax.broadcasted_iota(jnp.int32, (NB, G), 1)
    rowpos = lax.broadcasted_iota(jnp.int32, (NB, G), 0)
    valid = rowpos + i * NB < N
    goh = gid[:, None] == gi
    cb = counts.reshape(1, G)
    nn = jnp.sum(jnp.where(goh, cb, 0.0), axis=1)
    reward = jnp.where((nn == 1.0) & is42, -4.0, reward)

    selp_mean = selp / jnp.maximum(deg, 1.0)
    loss = -(nlp + selp_mean) * reward

    gohv = goh & valid
    ploss = jnp.sum(jnp.where(gohv, loss[:, None], 0.0), axis=0)
    prew = jnp.sum(jnp.where(gohv, reward[:, None], 0.0), axis=0)

    @pl.when(i == 0)
    def _():
        sl_acc[...] = jnp.zeros_like(sl_acc)
        sr_acc[...] = jnp.zeros_like(sr_acc)

    sl_acc[...] += ploss
    sr_acc[...] += prew

    @pl.when(i == NGRID - 1)
    def _():
        c = jnp.maximum(counts, 1.0)
        tl_ref[...] = (jnp.sum(sl_acc[...] / c) / G).reshape(1, 1)
        tr_ref[...] = (jnp.sum(sr_acc[...] / c) / G).reshape(1, 1)


def _finalize(sums8, is42, chg, arm, mxb, nlp, graph_ids, counts):
    return pl.pallas_call(
        _finalize_body,
        grid=(NGRID,),
        in_specs=[
            pl.BlockSpec((8, NB), lambda i: (0, i)),
            pl.BlockSpec((NB,), lambda i: (i,)),
            pl.BlockSpec((NB,), lambda i: (i,)),
            pl.BlockSpec((NB,), lambda i: (i,)),
            pl.BlockSpec((NB,), lambda i: (i,)),
            pl.BlockSpec((NB,), lambda i: (i,)),
            pl.BlockSpec((NB,), lambda i: (i,)),
            pl.BlockSpec((G,), lambda i: (0,)),
        ],
        out_specs=[
            pl.BlockSpec((1, 1), lambda i: (0, 0)),
            pl.BlockSpec((1, 1), lambda i: (0, 0)),
        ],
        out_shape=[jax.ShapeDtypeStruct((1, 1), jnp.float32) for _ in range(2)],
        scratch_shapes=[
            pltpu.VMEM((G,), jnp.float32),
            pltpu.VMEM((G,), jnp.float32),
        ],
    )(sums8, is42, chg, arm, mxb, nlp, graph_ids, counts)


@jax.jit
def kernel(x, edge_feats, node_logprobs, edge_logprobs, max_bonds_table,
           edge_index, isgen, graph_ids):
    tab_pad = jnp.zeros((D,), jnp.float32).at[:43].set(max_bonds_table)
    is42, chg, arm, mxb, counts = _node_feats(x, graph_ids, tab_pad)
    parts = _sc_edge_aggregate(edge_index, edge_logprobs, edge_feats, isgen,
                               is42)
    sums = parts.reshape(2, 4, NP)[:, :, :N].reshape(8, N)
    tl, tr = _finalize(sums, is42, chg, arm, mxb, node_logprobs, graph_ids,
                       counts)
    return (tl[0, 0], tr[0, 0])
